# baseline (speedup 1.0000x reference)
import functools

import jax
import jax.numpy as jnp
from jax import lax
from jax.experimental import pallas as pl
from jax.experimental.pallas import tpu as pltpu
from jax.experimental.pallas import tpu_sc as plsc

B = 16384
D = 64
NENT = 1000000
NREL = 1000
L = 16
CH = 128

_info = plsc.get_sparse_core_info()
NC, NS = _info.num_cores, _info.num_subcores
NW = NC * NS
BPW = B // NW
HALF = BPW // 2
NCH = HALF // CH
NGRP = HALF // L

_mesh = plsc.VectorSubcoreMesh(core_axis_name="c", subcore_axis_name="s")


@functools.partial(
    pl.kernel,
    mesh=_mesh,
    out_type=jax.ShapeDtypeStruct((B,), jnp.float32),
    compiler_params=pltpu.CompilerParams(needs_layout_passes=False),
    scratch_types=[
        pltpu.VMEM((BPW // CH, CH), jnp.int32),
        pltpu.VMEM((BPW // CH, CH), jnp.int32),
        pltpu.VMEM((BPW // CH, CH), jnp.int32),
        pltpu.VMEM((BPW // CH, CH), jnp.int32),
        pltpu.VMEM((BPW // CH, CH), jnp.int32),
        pltpu.VMEM((BPW // CH, CH), jnp.int32),
        pltpu.VMEM((HALF, 2 * D), jnp.float32),
        pltpu.VMEM((HALF, 2 * D), jnp.float32),
        pltpu.VMEM((HALF, 2 * D), jnp.float32),
        pltpu.VMEM((BPW,), jnp.float32),
        pltpu.SemaphoreType.DMA,
    ],
)
def _transe_sc(head_hbm, rel_hbm, tail_hbm, ent2_hbm, rel2_hbm, out_hbm,
               hi_v, ti_v, ri_v, hp_v, tp_v, rp_v, h_v, t_v, r_v, o_v, sem):
    wid = lax.axis_index("s") * NC + lax.axis_index("c")
    base = wid * BPW
    crow = wid * (BPW // CH)

    pltpu.sync_copy(head_hbm.at[pl.ds(crow, BPW // CH)], hi_v)
    pltpu.sync_copy(tail_hbm.at[pl.ds(crow, BPW // CH)], ti_v)
    pltpu.sync_copy(rel_hbm.at[pl.ds(crow, BPW // CH)], ri_v)

    def shift_rows(j, carry):
        hp_v[j // 8, pl.ds((j % 8) * L, L)] = (
            hi_v[j // 8, pl.ds((j % 8) * L, L)] >> 1)
        tp_v[j // 8, pl.ds((j % 8) * L, L)] = (
            ti_v[j // 8, pl.ds((j % 8) * L, L)] >> 1)
        rp_v[j // 8, pl.ds((j % 8) * L, L)] = (
            ri_v[j // 8, pl.ds((j % 8) * L, L)] >> 1)
        return carry

    for j in range(BPW // L):
        shift_rows(j, 0)

    lane = lax.iota(jnp.int32, L)

    for half in range(2):
        off = half * HALF

        copies = []
        for c in range(NCH):
            crow_l = half * NCH + c
            dst = pl.ds(c * CH, CH)
            copies.append(pltpu.async_copy(
                ent2_hbm.at[hp_v.at[crow_l]], h_v.at[dst], sem))
            copies.append(pltpu.async_copy(
                ent2_hbm.at[tp_v.at[crow_l]], t_v.at[dst], sem))
            copies.append(pltpu.async_copy(
                rel2_hbm.at[rp_v.at[crow_l]], r_v.at[dst], sem))
        for cp in copies:
            cp.wait()

        def group_body(g, carry):
            j0 = g * L
            crow_l = half * NCH
            hraw = hi_v[crow_l + g // 8, pl.ds((g % 8) * L, L)]
            traw = ti_v[crow_l + g // 8, pl.ds((g % 8) * L, L)]
            rraw = ri_v[crow_l + g // 8, pl.ds((g % 8) * L, L)]
            hoff = (hraw & 1) * D
            toff = (traw & 1) * D
            roff = (rraw & 1) * D
            rows = j0 + lane
            acc = jnp.zeros((L,), jnp.float32)
            for c in range(D):
                h = plsc.load_gather(h_v, [rows, hoff + c])
                t = plsc.load_gather(t_v, [rows, toff + c])
                r = plsc.load_gather(r_v, [rows, roff + c])
                acc = acc + jnp.abs(h + r - t)
            o_v[pl.ds(off + j0, L)] = acc
            return carry

        lax.fori_loop(0, NGRP, group_body, jnp.int32(0))

    pltpu.sync_copy(o_v, out_hbm.at[pl.ds(base, BPW)])


def kernel(head, relation, tail, entity_table, relation_table):
    head2 = head.reshape(B // CH, CH)
    rel2 = relation.reshape(B // CH, CH)
    tail2 = tail.reshape(B // CH, CH)
    ent_pairs = entity_table.reshape(NENT // 2, 2 * D)
    rel_pairs = relation_table.reshape(NREL // 2, 2 * D)
    return _transe_sc(head2, rel2, tail2, ent_pairs, rel_pairs)
